# baseline (speedup 1.0000x reference)
import functools

import jax
import jax.numpy as jnp
from jax import lax
from jax.experimental import pallas as pl
from jax.experimental.pallas import tpu as pltpu
from jax.experimental.pallas import tpu_sc as plsc

B, T, A = 64, 4096, 128
HIDDEN = 512
NEW = 16
FORCED = 8
KEEP_K = 1536
K_OLD = KEEP_K - FORCED
OLD_T = T - NEW
NVREG = T // 16
INF_BITS = 0x7F800000

_NC, _NS = 2, 16
_NW = _NC * _NS
_ROWS_PER_W = B // _NW

GCHUNK = 128
NCHUNK = KEEP_K // GCHUNK


def _gelu(x):
    return 0.5 * x * (1.0 + lax.erf(x * (2.0 ** -0.5)))


def _film_body(te_ref, sw1_ref, sb1_ref, sw2_ref, sb2_ref,
               hw1_ref, hb1_ref, hw2_ref, hb2_ref,
               g_ref, be_ref):
    te = te_ref[...]

    def mlp(w1, b1, w2, b2):
        h = _gelu(jnp.dot(te, w1, preferred_element_type=jnp.float32) + b1)
        return jnp.dot(h, w2, preferred_element_type=jnp.float32) + b2

    g_ref[...] = mlp(sw1_ref[...], sb1_ref[...], sw2_ref[...], sb2_ref[...])
    be_ref[...] = mlp(hw1_ref[...], hb1_ref[...], hw2_ref[...], hb2_ref[...])


def _film(text_embed, sw1, sb1, sw2, sb2, hw1, hb1, hw2, hb2):
    return pl.pallas_call(
        _film_body,
        out_shape=[jax.ShapeDtypeStruct((B, A), jnp.float32),
                   jax.ShapeDtypeStruct((B, A), jnp.float32)],
    )(text_embed, sw1, sb1, sw2, sb2, hw1, hb1, hw2, hb2)


def _scores(bank, g3, be3, rwt, rb, off, nrows):
    def body(bank_ref, g_ref, be_ref, rwt_ref, rb_ref, p_ref, th_ref,
             pall_ref):
        b = pl.program_id(0)
        x = bank_ref[0]
        film = x * (1.0 + g_ref[0]) + be_ref[0]
        lT = lax.dot_general(rwt_ref[...], film, (((1,), (1,)), ((), ())),
                             preferred_element_type=jnp.float32)
        l0 = lT[0:1, :] + rb_ref[0, 0]
        l1 = lT[1:2, :] + rb_ref[0, 1]
        m = jnp.maximum(l0, l1)
        e0 = jnp.exp(l0 - m)
        e1 = jnp.exp(l1 - m)
        p = e1 / (e0 + e1)
        p_ref[0, 0, :] = p[0]
        pall_ref[pl.ds(b, 1), :] = p

        @pl.when(b == nrows - 1)
        def _():
            bits = lax.bitcast_convert_type(pall_ref[...], jnp.int32)
            col = lax.broadcasted_iota(jnp.int32, (nrows, T), 1)
            obits = jnp.where(col < OLD_T, bits, -1)
            nbits = jnp.where(col >= OLD_T, bits, -1)
            th_o, r_o = _search(obits, K_OLD)
            th_n, r_n = _search(nbits, FORCED)
            pad = jnp.zeros((nrows, 12), jnp.int32)
            th_ref[...] = jnp.concatenate([th_o, r_o, th_n, r_n, pad],
                                          axis=1)

    return pl.pallas_call(
        body,
        grid=(nrows,),
        in_specs=[
            pl.BlockSpec((1, T, A), lambda b: (b + off, 0, 0)),
            pl.BlockSpec((1, 1, A), lambda b: (b + off, 0, 0)),
            pl.BlockSpec((1, 1, A), lambda b: (b + off, 0, 0)),
            pl.BlockSpec((2, A), lambda b: (0, 0)),
            pl.BlockSpec((1, 2), lambda b: (0, 0)),
        ],
        out_specs=[pl.BlockSpec((1, 1, T), lambda b: (b, 0, 0)),
                   pl.BlockSpec((nrows, 16), lambda b: (0, 0))],
        out_shape=[jax.ShapeDtypeStruct((nrows, 1, T), jnp.float32),
                   jax.ShapeDtypeStruct((nrows, 16), jnp.int32)],
        scratch_shapes=[pltpu.VMEM((nrows, T), jnp.float32)],
    )(bank, g3, be3, rwt, rb)


def _search(bmat, k):
    nr = bmat.shape[0]
    lo = jnp.zeros((nr, 1), jnp.int32)
    hi = jnp.full((nr, 1), INF_BITS, jnp.int32)

    def it(_, carry):
        lo, hi = carry
        mid = lo + lax.shift_right_logical(hi - lo, 1)
        cnt = jnp.sum((bmat >= mid).astype(jnp.int32), axis=1, keepdims=True)
        ge = cnt >= k
        return (jnp.where(ge, mid, lo), jnp.where(ge, hi, mid))

    lo, hi = lax.fori_loop(0, 31, it, (lo, hi))
    cgt = jnp.sum((bmat > lo).astype(jnp.int32), axis=1, keepdims=True)
    return lo, k - cgt


def _make_sc_body(base_row, rows_per_w):
    return functools.partial(_sc_body, base_row, rows_per_w)


def _sc_body(base_row, rows_per_w, probs_hbm, th_hbm, bank_hbm,
             seltok_hbm, selw_hbm,
             p_buf, th_buf, idx_buf, w_buf, tok0, tok1,
             gsem0, gsem1, wsem0, wsem1):
    wid = lax.axis_index("s") * _NC + lax.axis_index("c")
    lane = lax.iota(jnp.int32, 16)
    toks = (tok0, tok1)
    gsems = (gsem0, gsem1)
    wsems = (wsem0, wsem1)

    for j in range(rows_per_w):
        b = wid * rows_per_w + j
        pltpu.sync_copy(probs_hbm.at[b], p_buf)
        pltpu.sync_copy(th_hbm.at[b], th_buf)
        tv = th_buf[...]
        th_o = jnp.sum(jnp.where(lane == 0, tv, 0))
        r_o = jnp.sum(jnp.where(lane == 1, tv, 0))
        th_n = jnp.sum(jnp.where(lane == 2, tv, 0))
        r_n = jnp.sum(jnp.where(lane == 3, tv, 0))
        base = (b + base_row) * T

        def emit(i, off, tie, th, r):
            pv = p_buf[i]
            bits = plsc.bitcast(pv, jnp.int32)
            gt = bits > th
            eq = bits == th
            eqi = jnp.where(eq, 1, 0).astype(jnp.int32)
            ranks = plsc.cumsum(eqi) + tie
            keep = gt | (eq & (ranks <= r))
            idxv = lane + (base + i * 16)
            plsc.store_compressed(idx_buf.at[pl.ds(off, 16)], idxv, mask=keep)
            plsc.store_compressed(w_buf.at[pl.ds(off, 16)], pv, mask=keep)
            npop = jnp.sum(jnp.where(keep, 1, 0).astype(jnp.int32))
            neq = jnp.sum(eqi)
            return off + npop, tie + neq

        def body(i, carry):
            off, tie = carry
            return emit(i, off, tie, th_o, r_o)

        off, _ = lax.fori_loop(0, NVREG - 1, body,
                               (jnp.int32(0), jnp.int32(0)))
        emit(NVREG - 1, off, jnp.int32(0), th_n, r_n)

        pltpu.sync_copy(w_buf.at[pl.ds(0, KEEP_K)], selw_hbm.at[b])

        ghandles = [None, None]
        whandles = [None, None]

        def gstart(c):
            s = c % 2
            ghandles[s] = pltpu.async_copy(
                bank_hbm.at[idx_buf.at[pl.ds(c * GCHUNK, GCHUNK)]],
                toks[s], gsems[s])

        gstart(0)
        for c in range(NCHUNK):
            s = c % 2
            if c + 1 < NCHUNK:
                if c >= 1:
                    whandles[(c + 1) % 2].wait()
                gstart(c + 1)
            ghandles[s].wait()
            whandles[s] = pltpu.async_copy(
                toks[s], seltok_hbm.at[b, pl.ds(c * GCHUNK, GCHUNK)],
                wsems[s])
        whandles[0].wait()
        whandles[1].wait()


def _sc_select(probs2, th, bank_flat, base_row=0):
    nrows = probs2.shape[0]
    mesh = plsc.VectorSubcoreMesh(core_axis_name="c", subcore_axis_name="s",
                                  num_cores=_NC, num_subcores=_NS)
    f = pl.kernel(
        _make_sc_body(base_row, nrows // _NW),
        out_type=[jax.ShapeDtypeStruct((nrows, KEEP_K, A), jnp.float32),
                  jax.ShapeDtypeStruct((nrows, KEEP_K), jnp.float32)],
        mesh=mesh,
        compiler_params=pltpu.CompilerParams(needs_layout_passes=False),
        scratch_types=[
            pltpu.VMEM((NVREG, 16), jnp.float32),
            pltpu.VMEM((16,), jnp.int32),
            pltpu.VMEM((KEEP_K + 16,), jnp.int32),
            pltpu.VMEM((KEEP_K + 16,), jnp.float32),
            pltpu.VMEM((GCHUNK, A), jnp.float32),
            pltpu.VMEM((GCHUNK, A), jnp.float32),
            pltpu.SemaphoreType.DMA,
            pltpu.SemaphoreType.DMA,
            pltpu.SemaphoreType.DMA,
            pltpu.SemaphoreType.DMA,
        ],
    )
    return f(probs2, th, bank_flat)


TBLK = 1536


def _ffn_body(tok_ref, w_ref, lng_ref, lnb_ref,
              w1_ref, b1_ref, w2_ref, b2_ref, out_ref):
    x = tok_ref[0]
    mu = jnp.mean(x, axis=-1, keepdims=True)
    d = x - mu
    var = jnp.mean(d * d, axis=-1, keepdims=True)
    nrm = d * lax.rsqrt(var + 1e-5) * lng_ref[...] + lnb_ref[...]
    h = _gelu(jnp.dot(nrm.astype(jnp.bfloat16), w1_ref[...],
                      preferred_element_type=jnp.float32) + b1_ref[...])
    o = jnp.dot(h.astype(jnp.bfloat16), w2_ref[...],
                preferred_element_type=jnp.float32) + b2_ref[...]
    wcol = jnp.reshape(w_ref[0], (TBLK, 1))
    out_ref[0] = x + o * wcol


def _ffn(tok3, w3, ln_g, ln_b, w1, b1, w2, b2):
    ngrid = tok3.shape[0]
    return pl.pallas_call(
        _ffn_body,
        grid=(ngrid,),
        in_specs=[
            pl.BlockSpec((1, TBLK, A), lambda g: (g, 0, 0)),
            pl.BlockSpec((1, 1, TBLK), lambda g: (g, 0, 0)),
            pl.BlockSpec((1, A), lambda g: (0, 0)),
            pl.BlockSpec((1, A), lambda g: (0, 0)),
            pl.BlockSpec((A, 4 * A), lambda g: (0, 0)),
            pl.BlockSpec((1, 4 * A), lambda g: (0, 0)),
            pl.BlockSpec((4 * A, A), lambda g: (0, 0)),
            pl.BlockSpec((1, A), lambda g: (0, 0)),
        ],
        out_specs=pl.BlockSpec((1, TBLK, A), lambda g: (g, 0, 0)),
        out_shape=jax.ShapeDtypeStruct((ngrid, TBLK, A), jnp.float32),
    )(tok3, w3, ln_g, ln_b, w1.astype(jnp.bfloat16), b1,
      w2.astype(jnp.bfloat16), b2)


def kernel(new_action, text_embed, scale_w1, scale_b1, scale_w2, scale_b2,
           shift_w1, shift_b1, shift_w2, shift_b2, router_w, router_b,
           ln_g, ln_b, ffn_w1, ffn_b1, ffn_w2, ffn_b2):
    gamma, beta = _film(text_embed,
                        scale_w1, scale_b1.reshape(1, HIDDEN),
                        scale_w2, scale_b2.reshape(1, A),
                        shift_w1, shift_b1.reshape(1, HIDDEN),
                        shift_w2, shift_b2.reshape(1, A))

    g3 = gamma.reshape(B, 1, A)
    be3 = beta.reshape(B, 1, A)
    rwt = router_w.T
    rb2 = router_b.reshape(1, 2)
    bank_flat = new_action.reshape(B * T, A)
    half = B // 2

    outs = []
    for off in (0, half):
        p3, th = _scores(new_action, g3, be3, rwt, rb2, off, half)
        sel_tok, sel_w = _sc_select(p3.reshape(half, NVREG, 16), th,
                                    bank_flat, base_row=off)
        ngrid = half * (KEEP_K // TBLK)
        out = _ffn(sel_tok.reshape(ngrid, TBLK, A),
                   sel_w.reshape(ngrid, 1, TBLK),
                   ln_g.reshape(1, A), ln_b.reshape(1, A),
                   ffn_w1, ffn_b1.reshape(1, 4 * A),
                   ffn_w2, ffn_b2.reshape(1, A))
        outs.append(out.reshape(half, KEEP_K, A))
    return jnp.concatenate(outs, axis=0)


# speedup vs baseline: 1.1179x; 1.1179x over previous
import functools

import jax
import jax.numpy as jnp
from jax import lax
from jax.experimental import pallas as pl
from jax.experimental.pallas import tpu as pltpu
from jax.experimental.pallas import tpu_sc as plsc

B, T, A = 64, 4096, 128
HIDDEN = 512
NEW = 16
FORCED = 8
KEEP_K = 1536
K_OLD = KEEP_K - FORCED
OLD_T = T - NEW
NVREG = T // 16
INF_BITS = 0x7F800000

_NC, _NS = 2, 16
_NW = _NC * _NS
_ROWS_PER_W = B // _NW

GCHUNK = 128
NCHUNK = KEEP_K // GCHUNK


def _gelu(x):
    return 0.5 * x * (1.0 + lax.erf(x * (2.0 ** -0.5)))


def _film_body(te_ref, sw1_ref, sb1_ref, sw2_ref, sb2_ref,
               hw1_ref, hb1_ref, hw2_ref, hb2_ref,
               g_ref, be_ref):
    te = te_ref[...]

    def mlp(w1, b1, w2, b2):
        h = _gelu(jnp.dot(te, w1, preferred_element_type=jnp.float32) + b1)
        return jnp.dot(h, w2, preferred_element_type=jnp.float32) + b2

    g_ref[...] = mlp(sw1_ref[...], sb1_ref[...], sw2_ref[...], sb2_ref[...])
    be_ref[...] = mlp(hw1_ref[...], hb1_ref[...], hw2_ref[...], hb2_ref[...])


def _film(text_embed, sw1, sb1, sw2, sb2, hw1, hb1, hw2, hb2):
    return pl.pallas_call(
        _film_body,
        out_shape=[jax.ShapeDtypeStruct((B, A), jnp.float32),
                   jax.ShapeDtypeStruct((B, A), jnp.float32)],
    )(text_embed, sw1, sb1, sw2, sb2, hw1, hb1, hw2, hb2)


def _scores(bank, g3, be3, rwt, rb, off, nrows):
    def body(bank_ref, g_ref, be_ref, rwt_ref, rb_ref, p_ref, th_ref,
             pall_ref):
        b = pl.program_id(0)
        x = bank_ref[0]
        film = x * (1.0 + g_ref[0]) + be_ref[0]
        lT = lax.dot_general(rwt_ref[...], film, (((1,), (1,)), ((), ())),
                             preferred_element_type=jnp.float32)
        l0 = lT[0:1, :] + rb_ref[0, 0]
        l1 = lT[1:2, :] + rb_ref[0, 1]
        m = jnp.maximum(l0, l1)
        e0 = jnp.exp(l0 - m)
        e1 = jnp.exp(l1 - m)
        p = e1 / (e0 + e1)
        p_ref[0, 0, :] = p[0]
        pall_ref[pl.ds(b, 1), :] = p

        @pl.when(b == nrows - 1)
        def _():
            bits = lax.bitcast_convert_type(pall_ref[...], jnp.int32)
            col = lax.broadcasted_iota(jnp.int32, (nrows, T), 1)
            obits = jnp.where(col < OLD_T, bits, -1)
            nbits = jnp.where(col >= OLD_T, bits, -1)
            th_o, r_o = _search(obits, K_OLD)
            th_n, r_n = _search(nbits, FORCED)
            pad = jnp.zeros((nrows, 12), jnp.int32)
            th_ref[...] = jnp.concatenate([th_o, r_o, th_n, r_n, pad],
                                          axis=1)

    return pl.pallas_call(
        body,
        grid=(nrows,),
        in_specs=[
            pl.BlockSpec((1, T, A), lambda b: (b + off, 0, 0)),
            pl.BlockSpec((1, 1, A), lambda b: (b + off, 0, 0)),
            pl.BlockSpec((1, 1, A), lambda b: (b + off, 0, 0)),
            pl.BlockSpec((2, A), lambda b: (0, 0)),
            pl.BlockSpec((1, 2), lambda b: (0, 0)),
        ],
        out_specs=[pl.BlockSpec((1, 1, T), lambda b: (b, 0, 0)),
                   pl.BlockSpec((nrows, 16), lambda b: (0, 0))],
        out_shape=[jax.ShapeDtypeStruct((nrows, 1, T), jnp.float32),
                   jax.ShapeDtypeStruct((nrows, 16), jnp.int32)],
        scratch_shapes=[pltpu.VMEM((nrows, T), jnp.float32)],
    )(bank, g3, be3, rwt, rb)


def _search(bmat, k):
    nr = bmat.shape[0]
    lo = jnp.zeros((nr, 1), jnp.int32)
    hi = jnp.full((nr, 1), INF_BITS, jnp.int32)

    def it(_, carry):
        lo, hi = carry
        mid = lo + lax.shift_right_logical(hi - lo, 1)
        cnt = jnp.sum((bmat >= mid).astype(jnp.int32), axis=1, keepdims=True)
        ge = cnt >= k
        return (jnp.where(ge, mid, lo), jnp.where(ge, hi, mid))

    lo, hi = lax.fori_loop(0, 31, it, (lo, hi))
    cgt = jnp.sum((bmat > lo).astype(jnp.int32), axis=1, keepdims=True)
    return lo, k - cgt


def _make_sc_body(base_row, rows_per_w):
    return functools.partial(_sc_body, base_row, rows_per_w)


def _sc_body(base_row, rows_per_w, probs_hbm, th_hbm, bank_hbm,
             seltok_hbm, selw_hbm,
             p_buf, th_buf, idx_buf, w_buf, tok0, tok1,
             gsem0, gsem1, wsem0, wsem1):
    wid = lax.axis_index("s") * _NC + lax.axis_index("c")
    lane = lax.iota(jnp.int32, 16)
    toks = (tok0, tok1)
    gsems = (gsem0, gsem1)
    wsems = (wsem0, wsem1)

    for j in range(rows_per_w):
        b = wid * rows_per_w + j
        pltpu.sync_copy(probs_hbm.at[b], p_buf)
        pltpu.sync_copy(th_hbm.at[b], th_buf)
        tv = th_buf[...]
        th_o = jnp.sum(jnp.where(lane == 0, tv, 0))
        r_o = jnp.sum(jnp.where(lane == 1, tv, 0))
        th_n = jnp.sum(jnp.where(lane == 2, tv, 0))
        r_n = jnp.sum(jnp.where(lane == 3, tv, 0))
        base = (b + base_row) * T

        def emit(i, off, tie, th, r):
            pv = p_buf[i]
            bits = plsc.bitcast(pv, jnp.int32)
            gt = bits > th
            eq = bits == th
            eqi = jnp.where(eq, 1, 0).astype(jnp.int32)
            ranks = plsc.cumsum(eqi) + tie
            keep = gt | (eq & (ranks <= r))
            idxv = lane + (base + i * 16)
            plsc.store_compressed(idx_buf.at[pl.ds(off, 16)], idxv, mask=keep)
            plsc.store_compressed(w_buf.at[pl.ds(off, 16)], pv, mask=keep)
            npop = jnp.sum(jnp.where(keep, 1, 0).astype(jnp.int32))
            neq = jnp.sum(eqi)
            return off + npop, tie + neq

        def body(i, carry):
            off, tie = carry
            return emit(i, off, tie, th_o, r_o)

        off, _ = lax.fori_loop(0, NVREG - 1, body,
                               (jnp.int32(0), jnp.int32(0)))
        emit(NVREG - 1, off, jnp.int32(0), th_n, r_n)

        pltpu.sync_copy(w_buf.at[pl.ds(0, KEEP_K)], selw_hbm.at[b])

        ghandles = [None, None]
        whandles = [None, None]

        def gstart(c):
            s = c % 2
            ghandles[s] = pltpu.async_copy(
                bank_hbm.at[idx_buf.at[pl.ds(c * GCHUNK, GCHUNK)]],
                toks[s], gsems[s])

        gstart(0)
        for c in range(NCHUNK):
            s = c % 2
            if c + 1 < NCHUNK:
                if c >= 1:
                    whandles[(c + 1) % 2].wait()
                gstart(c + 1)
            ghandles[s].wait()
            whandles[s] = pltpu.async_copy(
                toks[s], seltok_hbm.at[b, pl.ds(c * GCHUNK, GCHUNK)],
                wsems[s])
        whandles[0].wait()
        whandles[1].wait()


def _sc_select(probs2, th, bank_flat, base_row=0):
    nrows = probs2.shape[0]
    mesh = plsc.VectorSubcoreMesh(core_axis_name="c", subcore_axis_name="s",
                                  num_cores=_NC, num_subcores=_NS)
    f = pl.kernel(
        _make_sc_body(base_row, nrows // _NW),
        out_type=[jax.ShapeDtypeStruct((nrows, KEEP_K, A), jnp.float32),
                  jax.ShapeDtypeStruct((nrows, KEEP_K), jnp.float32)],
        mesh=mesh,
        compiler_params=pltpu.CompilerParams(needs_layout_passes=False),
        scratch_types=[
            pltpu.VMEM((NVREG, 16), jnp.float32),
            pltpu.VMEM((16,), jnp.int32),
            pltpu.VMEM((KEEP_K + 16,), jnp.int32),
            pltpu.VMEM((KEEP_K + 16,), jnp.float32),
            pltpu.VMEM((GCHUNK, A), jnp.float32),
            pltpu.VMEM((GCHUNK, A), jnp.float32),
            pltpu.SemaphoreType.DMA,
            pltpu.SemaphoreType.DMA,
            pltpu.SemaphoreType.DMA,
            pltpu.SemaphoreType.DMA,
        ],
    )
    return f(probs2, th, bank_flat)


TBLK = 1536


def _ffn_body(tok_ref, w_ref, lng_ref, lnb_ref,
              w1_ref, b1_ref, w2_ref, b2_ref, out_ref):
    x = tok_ref[0]
    mu = jnp.mean(x, axis=-1, keepdims=True)
    d = x - mu
    var = jnp.mean(d * d, axis=-1, keepdims=True)
    nrm = d * lax.rsqrt(var + 1e-5) * lng_ref[...] + lnb_ref[...]
    h = _gelu(jnp.dot(nrm.astype(jnp.bfloat16), w1_ref[...],
                      preferred_element_type=jnp.float32) + b1_ref[...])
    o = jnp.dot(h.astype(jnp.bfloat16), w2_ref[...],
                preferred_element_type=jnp.float32) + b2_ref[...]
    wcol = jnp.reshape(w_ref[0], (TBLK, 1))
    out_ref[0] = x + o * wcol


def _ffn(tok3, w3, ln_g, ln_b, w1, b1, w2, b2):
    ngrid = tok3.shape[0]
    return pl.pallas_call(
        _ffn_body,
        grid=(ngrid,),
        in_specs=[
            pl.BlockSpec((1, TBLK, A), lambda g: (g, 0, 0)),
            pl.BlockSpec((1, 1, TBLK), lambda g: (g, 0, 0)),
            pl.BlockSpec((1, A), lambda g: (0, 0)),
            pl.BlockSpec((1, A), lambda g: (0, 0)),
            pl.BlockSpec((A, 4 * A), lambda g: (0, 0)),
            pl.BlockSpec((1, 4 * A), lambda g: (0, 0)),
            pl.BlockSpec((4 * A, A), lambda g: (0, 0)),
            pl.BlockSpec((1, A), lambda g: (0, 0)),
        ],
        out_specs=pl.BlockSpec((1, TBLK, A), lambda g: (g, 0, 0)),
        out_shape=jax.ShapeDtypeStruct((ngrid, TBLK, A), jnp.float32),
    )(tok3, w3, ln_g, ln_b, w1.astype(jnp.bfloat16), b1,
      w2.astype(jnp.bfloat16), b2)


def kernel(new_action, text_embed, scale_w1, scale_b1, scale_w2, scale_b2,
           shift_w1, shift_b1, shift_w2, shift_b2, router_w, router_b,
           ln_g, ln_b, ffn_w1, ffn_b1, ffn_w2, ffn_b2):
    gamma, beta = _film(text_embed,
                        scale_w1, scale_b1.reshape(1, HIDDEN),
                        scale_w2, scale_b2.reshape(1, A),
                        shift_w1, shift_b1.reshape(1, HIDDEN),
                        shift_w2, shift_b2.reshape(1, A))

    p3, th = _scores(new_action, gamma.reshape(B, 1, A),
                     beta.reshape(B, 1, A),
                     router_w.T, router_b.reshape(1, 2), 0, B)
    sel_tok, sel_w = _sc_select(p3.reshape(B, NVREG, 16), th,
                                new_action.reshape(B * T, A))
    ngrid = B * (KEEP_K // TBLK)
    out = _ffn(sel_tok.reshape(ngrid, TBLK, A),
               sel_w.reshape(ngrid, 1, TBLK),
               ln_g.reshape(1, A), ln_b.reshape(1, A),
               ffn_w1, ffn_b1.reshape(1, 4 * A),
               ffn_w2, ffn_b2.reshape(1, A))
    return out.reshape(B, KEEP_K, A)


# speedup vs baseline: 1.1191x; 1.0010x over previous
import functools

import jax
import jax.numpy as jnp
from jax import lax
from jax.experimental import pallas as pl
from jax.experimental.pallas import tpu as pltpu
from jax.experimental.pallas import tpu_sc as plsc

B, T, A = 64, 4096, 128
HIDDEN = 512
D_TEXT = 768
NEW = 16
FORCED = 8
KEEP_K = 1536
K_OLD = KEEP_K - FORCED
OLD_T = T - NEW
NVREG = T // 16
INF_BITS = 0x7F800000

_NC, _NS = 2, 16
_NW = _NC * _NS
_ROWS_PER_W = B // _NW

GCHUNK = 128
NCHUNK = KEEP_K // GCHUNK


def _gelu(x):
    return 0.5 * x * (1.0 + lax.erf(x * (2.0 ** -0.5)))


def _scores(bank, te, sw1, sb1, sw2, sb2, hw1, hb1, hw2, hb2,
            rwt, rb, off, nrows):
    def body(bank_ref, te_ref, sw1_ref, sb1_ref, sw2_ref, sb2_ref,
             hw1_ref, hb1_ref, hw2_ref, hb2_ref, rwt_ref, rb_ref,
             p_ref, th_ref, pall_ref, g_all, be_all):
        b = pl.program_id(0)

        @pl.when(b == 0)
        def _():
            te = te_ref[...]

            def mlp(w1, b1, w2, b2):
                h = _gelu(jnp.dot(te, w1,
                                  preferred_element_type=jnp.float32) + b1)
                return jnp.dot(h, w2,
                               preferred_element_type=jnp.float32) + b2

            g_all[...] = mlp(sw1_ref[...], sb1_ref[...], sw2_ref[...],
                             sb2_ref[...])
            be_all[...] = mlp(hw1_ref[...], hb1_ref[...], hw2_ref[...],
                              hb2_ref[...])

        x = bank_ref[0]
        g = g_all[pl.ds(b + off, 1), :]
        be = be_all[pl.ds(b + off, 1), :]
        film = x * (1.0 + g) + be
        lT = lax.dot_general(rwt_ref[...], film, (((1,), (1,)), ((), ())),
                             preferred_element_type=jnp.float32)
        l0 = lT[0:1, :] + rb_ref[0, 0]
        l1 = lT[1:2, :] + rb_ref[0, 1]
        m = jnp.maximum(l0, l1)
        e0 = jnp.exp(l0 - m)
        e1 = jnp.exp(l1 - m)
        p = e1 / (e0 + e1)
        p_ref[0, 0, :] = p[0]
        pall_ref[pl.ds(b, 1), :] = p

        @pl.when(b == nrows - 1)
        def _():
            bits = lax.bitcast_convert_type(pall_ref[...], jnp.int32)
            col = lax.broadcasted_iota(jnp.int32, (nrows, T), 1)
            obits = jnp.where(col < OLD_T, bits, -1)
            nbits = jnp.where(col >= OLD_T, bits, -1)
            th_o, r_o = _search(obits, K_OLD)
            th_n, r_n = _search(nbits, FORCED)
            pad = jnp.zeros((nrows, 12), jnp.int32)
            th_ref[...] = jnp.concatenate([th_o, r_o, th_n, r_n, pad],
                                          axis=1)

    full = lambda *s: pl.BlockSpec(s, lambda b: (0,) * len(s))
    return pl.pallas_call(
        body,
        grid=(nrows,),
        in_specs=[
            pl.BlockSpec((1, T, A), lambda b: (b + off, 0, 0)),
            full(B, D_TEXT),
            full(D_TEXT, HIDDEN), full(1, HIDDEN), full(HIDDEN, A), full(1, A),
            full(D_TEXT, HIDDEN), full(1, HIDDEN), full(HIDDEN, A), full(1, A),
            full(2, A),
            full(1, 2),
        ],
        out_specs=[pl.BlockSpec((1, 1, T), lambda b: (b, 0, 0)),
                   pl.BlockSpec((nrows, 16), lambda b: (0, 0))],
        out_shape=[jax.ShapeDtypeStruct((nrows, 1, T), jnp.float32),
                   jax.ShapeDtypeStruct((nrows, 16), jnp.int32)],
        scratch_shapes=[pltpu.VMEM((nrows, T), jnp.float32),
                        pltpu.VMEM((B, A), jnp.float32),
                        pltpu.VMEM((B, A), jnp.float32)],
    )(bank, te, sw1, sb1, sw2, sb2, hw1, hb1, hw2, hb2, rwt, rb)


def _search(bmat, k):
    nr = bmat.shape[0]
    lo = jnp.zeros((nr, 1), jnp.int32)
    hi = jnp.full((nr, 1), INF_BITS, jnp.int32)

    def it(_, carry):
        lo, hi = carry
        mid = lo + lax.shift_right_logical(hi - lo, 1)
        cnt = jnp.sum((bmat >= mid).astype(jnp.int32), axis=1, keepdims=True)
        ge = cnt >= k
        return (jnp.where(ge, mid, lo), jnp.where(ge, hi, mid))

    lo, hi = lax.fori_loop(0, 31, it, (lo, hi))
    cgt = jnp.sum((bmat > lo).astype(jnp.int32), axis=1, keepdims=True)
    return lo, k - cgt


def _make_sc_body(base_row, rows_per_w):
    return functools.partial(_sc_body, base_row, rows_per_w)


def _sc_body(base_row, rows_per_w, probs_hbm, th_hbm, bank_hbm,
             seltok_hbm, selw_hbm,
             p_buf, th_buf, idx_buf, w_buf, tok0, tok1,
             gsem0, gsem1, wsem0, wsem1):
    wid = lax.axis_index("s") * _NC + lax.axis_index("c")
    lane = lax.iota(jnp.int32, 16)
    toks = (tok0, tok1)
    gsems = (gsem0, gsem1)
    wsems = (wsem0, wsem1)

    for j in range(rows_per_w):
        b = wid * rows_per_w + j
        pltpu.sync_copy(probs_hbm.at[b], p_buf)
        pltpu.sync_copy(th_hbm.at[b], th_buf)
        tv = th_buf[...]
        th_o = jnp.sum(jnp.where(lane == 0, tv, 0))
        r_o = jnp.sum(jnp.where(lane == 1, tv, 0))
        th_n = jnp.sum(jnp.where(lane == 2, tv, 0))
        r_n = jnp.sum(jnp.where(lane == 3, tv, 0))
        base = (b + base_row) * T

        def emit(i, off, tie, th, r):
            pv = p_buf[i]
            bits = plsc.bitcast(pv, jnp.int32)
            gt = bits > th
            eq = bits == th
            eqi = jnp.where(eq, 1, 0).astype(jnp.int32)
            ranks = plsc.cumsum(eqi) + tie
            keep = gt | (eq & (ranks <= r))
            idxv = lane + (base + i * 16)
            plsc.store_compressed(idx_buf.at[pl.ds(off, 16)], idxv, mask=keep)
            plsc.store_compressed(w_buf.at[pl.ds(off, 16)], pv, mask=keep)
            npop = jnp.sum(jnp.where(keep, 1, 0).astype(jnp.int32))
            neq = jnp.sum(eqi)
            return off + npop, tie + neq

        def body(i, carry):
            off, tie = carry
            return emit(i, off, tie, th_o, r_o)

        off, _ = lax.fori_loop(0, NVREG - 1, body,
                               (jnp.int32(0), jnp.int32(0)))
        emit(NVREG - 1, off, jnp.int32(0), th_n, r_n)

        pltpu.sync_copy(w_buf.at[pl.ds(0, KEEP_K)], selw_hbm.at[b])

        ghandles = [None, None]
        whandles = [None, None]

        def gstart(c):
            s = c % 2
            ghandles[s] = pltpu.async_copy(
                bank_hbm.at[idx_buf.at[pl.ds(c * GCHUNK, GCHUNK)]],
                toks[s], gsems[s])

        gstart(0)
        for c in range(NCHUNK):
            s = c % 2
            if c + 1 < NCHUNK:
                if c >= 1:
                    whandles[(c + 1) % 2].wait()
                gstart(c + 1)
            ghandles[s].wait()
            whandles[s] = pltpu.async_copy(
                toks[s], seltok_hbm.at[b, pl.ds(c * GCHUNK, GCHUNK)],
                wsems[s])
        whandles[0].wait()
        whandles[1].wait()


def _sc_select(probs2, th, bank_flat, base_row=0):
    nrows = probs2.shape[0]
    mesh = plsc.VectorSubcoreMesh(core_axis_name="c", subcore_axis_name="s",
                                  num_cores=_NC, num_subcores=_NS)
    f = pl.kernel(
        _make_sc_body(base_row, nrows // _NW),
        out_type=[jax.ShapeDtypeStruct((nrows, KEEP_K, A), jnp.float32),
                  jax.ShapeDtypeStruct((nrows, KEEP_K), jnp.float32)],
        mesh=mesh,
        compiler_params=pltpu.CompilerParams(needs_layout_passes=False),
        scratch_types=[
            pltpu.VMEM((NVREG, 16), jnp.float32),
            pltpu.VMEM((16,), jnp.int32),
            pltpu.VMEM((KEEP_K + 16,), jnp.int32),
            pltpu.VMEM((KEEP_K + 16,), jnp.float32),
            pltpu.VMEM((GCHUNK, A), jnp.float32),
            pltpu.VMEM((GCHUNK, A), jnp.float32),
            pltpu.SemaphoreType.DMA,
            pltpu.SemaphoreType.DMA,
            pltpu.SemaphoreType.DMA,
            pltpu.SemaphoreType.DMA,
        ],
    )
    return f(probs2, th, bank_flat)


TBLK = 1536


def _ffn_body(tok_ref, w_ref, lng_ref, lnb_ref,
              w1_ref, b1_ref, w2_ref, b2_ref, out_ref):
    x = tok_ref[0]
    mu = jnp.mean(x, axis=-1, keepdims=True)
    d = x - mu
    var = jnp.mean(d * d, axis=-1, keepdims=True)
    nrm = d * lax.rsqrt(var + 1e-5) * lng_ref[...] + lnb_ref[...]
    h = _gelu(jnp.dot(nrm.astype(jnp.bfloat16), w1_ref[...],
                      preferred_element_type=jnp.float32) + b1_ref[...])
    o = jnp.dot(h.astype(jnp.bfloat16), w2_ref[...],
                preferred_element_type=jnp.float32) + b2_ref[...]
    wcol = jnp.reshape(w_ref[0], (TBLK, 1))
    out_ref[0] = x + o * wcol


def _ffn(tok3, w3, ln_g, ln_b, w1, b1, w2, b2):
    ngrid = tok3.shape[0]
    return pl.pallas_call(
        _ffn_body,
        grid=(ngrid,),
        in_specs=[
            pl.BlockSpec((1, TBLK, A), lambda g: (g, 0, 0)),
            pl.BlockSpec((1, 1, TBLK), lambda g: (g, 0, 0)),
            pl.BlockSpec((1, A), lambda g: (0, 0)),
            pl.BlockSpec((1, A), lambda g: (0, 0)),
            pl.BlockSpec((A, 4 * A), lambda g: (0, 0)),
            pl.BlockSpec((1, 4 * A), lambda g: (0, 0)),
            pl.BlockSpec((4 * A, A), lambda g: (0, 0)),
            pl.BlockSpec((1, A), lambda g: (0, 0)),
        ],
        out_specs=pl.BlockSpec((1, TBLK, A), lambda g: (g, 0, 0)),
        out_shape=jax.ShapeDtypeStruct((ngrid, TBLK, A), jnp.float32),
    )(tok3, w3, ln_g, ln_b, w1.astype(jnp.bfloat16), b1,
      w2.astype(jnp.bfloat16), b2)


def kernel(new_action, text_embed, scale_w1, scale_b1, scale_w2, scale_b2,
           shift_w1, shift_b1, shift_w2, shift_b2, router_w, router_b,
           ln_g, ln_b, ffn_w1, ffn_b1, ffn_w2, ffn_b2):
    p3, th = _scores(new_action, text_embed,
                     scale_w1, scale_b1.reshape(1, HIDDEN),
                     scale_w2, scale_b2.reshape(1, A),
                     shift_w1, shift_b1.reshape(1, HIDDEN),
                     shift_w2, shift_b2.reshape(1, A),
                     router_w.T, router_b.reshape(1, 2), 0, B)
    sel_tok, sel_w = _sc_select(p3.reshape(B, NVREG, 16), th,
                                new_action.reshape(B * T, A))
    ngrid = B * (KEEP_K // TBLK)
    out = _ffn(sel_tok.reshape(ngrid, TBLK, A),
               sel_w.reshape(ngrid, 1, TBLK),
               ln_g.reshape(1, A), ln_b.reshape(1, A),
               ffn_w1, ffn_b1.reshape(1, 4 * A),
               ffn_w2, ffn_b2.reshape(1, A))
    return out.reshape(B, KEEP_K, A)
